# baseline (speedup 1.0000x reference)
import jax
import jax.numpy as jnp
from jax import lax
from jax.experimental import pallas as pl
from jax.experimental.pallas import tpu as pltpu
from jax.experimental.pallas import tpu_sc as plsc

_N = 10000
_E = 320000
_D = 128
_G = 16
_EPS = 1e-5

_R = 1000
_NB = _N // _R

_NC = 2
_NS = 16
_CH = 125
_EPT = _E // (_NC * _NS)
_NCHUNK = _EPT // _CH
_NP = 10240
_RPT = _NP // _NS


def _msg_body(table, src2, dst2, zeros, out,
              idx_s, idx_d, rows, acc, sem):
    c = lax.axis_index("c")
    s = lax.axis_index("s")
    w = s * _NC + c
    pltpu.sync_copy(zeros.at[pl.ds(s * _RPT, _RPT)],
                    acc.at[pl.ds(s * _RPT, _RPT)])
    pltpu.sync_copy(src2.at[w], idx_s)
    pltpu.sync_copy(dst2.at[w], idx_d)
    plsc.subcore_barrier()

    def chunk(i, carry):
        pltpu.async_copy(table.at[idx_s.at[i]], rows, sem).wait()
        pltpu.sync_copy(rows, acc.at[idx_d.at[i]], add=True)
        return carry

    lax.fori_loop(0, _NCHUNK, chunk, 0)
    plsc.subcore_barrier()
    pltpu.sync_copy(acc.at[pl.ds(s * _RPT, _RPT)],
                    out.at[c, pl.ds(s * _RPT, _RPT)])


_sc_msg = pl.kernel(
    _msg_body,
    out_type=jax.ShapeDtypeStruct((_NC, _NP, _D), jnp.float32),
    mesh=plsc.VectorSubcoreMesh(core_axis_name="c", subcore_axis_name="s",
                                num_cores=_NC, num_subcores=_NS),
    scratch_types=[
        pltpu.VMEM((_NCHUNK, _CH), jnp.int32),
        pltpu.VMEM((_NCHUNK, _CH), jnp.int32),
        pltpu.VMEM((_CH, _D), jnp.float32),
        pltpu.VMEM_SHARED((_NP, _D), jnp.float32),
        pltpu.SemaphoreType.DMA,
    ],
)


_EB = 4000
_NQ = _NP // _D


def _deg_body(dst_ref, deg_ref):
    i = pl.program_id(0)
    d = dst_ref[...]
    ohq = (d // _D == lax.broadcasted_iota(jnp.int32, (_EB, _NQ), 1)
           ).astype(jnp.float32)
    ohr = (d % _D == lax.broadcasted_iota(jnp.int32, (_EB, _D), 1)
           ).astype(jnp.float32)

    @pl.when(i == 0)
    def _():
        deg_ref[...] = jnp.zeros_like(deg_ref)

    deg_ref[...] += lax.dot_general(ohq, ohr, (((0,), (0,)), ((), ())),
                                    preferred_element_type=jnp.float32)


_deg = pl.pallas_call(
    _deg_body,
    grid=(_E // _EB,),
    in_specs=[pl.BlockSpec((_EB, 1), lambda i: (i, 0))],
    out_specs=pl.BlockSpec((_NQ, _D), lambda i: (0, 0)),
    out_shape=jax.ShapeDtypeStruct((_NQ, _D), jnp.float32),
)


def _prep_body(x_ref, degp_ref, emb_ref, w0_ref, hh_ref, dinv_ref):
    deg = degp_ref[...] + 1.0
    dinv = lax.rsqrt(deg)
    oh = (x_ref[...] == lax.broadcasted_iota(jnp.int32, (_R, _D), 1))
    ew = jnp.dot(emb_ref[...], w0_ref[...], preferred_element_type=jnp.float32,
                 precision=lax.Precision.HIGHEST)
    hh_ref[...] = jnp.dot(oh.astype(jnp.float32), ew,
                          preferred_element_type=jnp.float32,
                 precision=lax.Precision.HIGHEST) * dinv
    dinv_ref[...] = dinv


_prep = pl.pallas_call(
    _prep_body,
    grid=(_NB,),
    in_specs=[
        pl.BlockSpec((_R, 1), lambda i: (i, 0)),
        pl.BlockSpec((_R, 1), lambda i: (i, 0)),
        pl.BlockSpec((_D, _D), lambda i: (0, 0)),
        pl.BlockSpec((_D, _D), lambda i: (0, 0)),
    ],
    out_specs=[
        pl.BlockSpec((_R, _D), lambda i: (i, 0)),
        pl.BlockSpec((_R, 1), lambda i: (i, 0)),
    ],
    out_shape=[
        jax.ShapeDtypeStruct((_N, _D), jnp.float32),
        jax.ShapeDtypeStruct((_N, 1), jnp.float32),
    ],
)


def _stats_body(part_ref, hh_ref, dinv_ref, b_ref, a_ref, st_ref):
    i = pl.program_id(0)
    act = (part_ref[0] + part_ref[1] + hh_ref[...]) * dinv_ref[...] + b_ref[...]
    a = jnp.maximum(act, 0.0)
    a_ref[...] = a

    @pl.when(i == 0)
    def _():
        st_ref[...] = jnp.zeros_like(st_ref)

    st_ref[0:1, :] += jnp.sum(a, axis=0, keepdims=True)
    st_ref[1:2, :] += jnp.sum(a * a, axis=0, keepdims=True)


_stats = pl.pallas_call(
    _stats_body,
    grid=(_NB,),
    in_specs=[
        pl.BlockSpec((_NC, _R, _D), lambda i: (0, i, 0)),
        pl.BlockSpec((_R, _D), lambda i: (i, 0)),
        pl.BlockSpec((_R, 1), lambda i: (i, 0)),
        pl.BlockSpec((1, _D), lambda i: (0, 0)),
    ],
    out_specs=[
        pl.BlockSpec((_R, _D), lambda i: (i, 0)),
        pl.BlockSpec((2, _D), lambda i: (0, 0)),
    ],
    out_shape=[
        jax.ShapeDtypeStruct((_N, _D), jnp.float32),
        jax.ShapeDtypeStruct((2, _D), jnp.float32),
    ],
)


def _bn(a_ref, st_ref, g_ref, be_ref):
    mu = st_ref[0:1, :] * (1.0 / _N)
    var = st_ref[1:2, :] * (1.0 / _N) - mu * mu
    return (a_ref[...] - mu) * lax.rsqrt(var + _EPS) * g_ref[...] + be_ref[...]


def _next_body(a_ref, st_ref, g_ref, be_ref, w_ref, dinv_ref, o_ref):
    hn = _bn(a_ref, st_ref, g_ref, be_ref)
    o_ref[...] = jnp.dot(hn, w_ref[...],
                         preferred_element_type=jnp.float32,
                 precision=lax.Precision.HIGHEST) * dinv_ref[...]


_next = pl.pallas_call(
    _next_body,
    grid=(_NB,),
    in_specs=[
        pl.BlockSpec((_R, _D), lambda i: (i, 0)),
        pl.BlockSpec((2, _D), lambda i: (0, 0)),
        pl.BlockSpec((1, _D), lambda i: (0, 0)),
        pl.BlockSpec((1, _D), lambda i: (0, 0)),
        pl.BlockSpec((_D, _D), lambda i: (0, 0)),
        pl.BlockSpec((_R, 1), lambda i: (i, 0)),
    ],
    out_specs=pl.BlockSpec((_R, _D), lambda i: (i, 0)),
    out_shape=jax.ShapeDtypeStruct((_N, _D), jnp.float32),
)


def _final_body(a_ref, st_ref, g_ref, be_ref, batch_ref, o_ref, sums, cnt):
    i = pl.program_id(0)
    hn = _bn(a_ref, st_ref, g_ref, be_ref)
    oh = (batch_ref[...] == lax.broadcasted_iota(jnp.int32, (_R, _G), 1)
          ).astype(jnp.float32)

    @pl.when(i == 0)
    def _():
        sums[...] = jnp.zeros_like(sums)
        cnt[...] = jnp.zeros_like(cnt)

    sums[...] += lax.dot_general(oh, hn, (((0,), (0,)), ((), ())),
                                 preferred_element_type=jnp.float32,
                                 precision=lax.Precision.HIGHEST)
    cnt[...] += jnp.sum(oh, axis=0)[:, None]

    @pl.when(i == _NB - 1)
    def _():
        o_ref[...] = sums[...] / jnp.maximum(cnt[...], 1.0)


_final = pl.pallas_call(
    _final_body,
    grid=(_NB,),
    in_specs=[
        pl.BlockSpec((_R, _D), lambda i: (i, 0)),
        pl.BlockSpec((2, _D), lambda i: (0, 0)),
        pl.BlockSpec((1, _D), lambda i: (0, 0)),
        pl.BlockSpec((1, _D), lambda i: (0, 0)),
        pl.BlockSpec((_R, 1), lambda i: (i, 0)),
    ],
    out_specs=pl.BlockSpec((_G, _D), lambda i: (0, 0)),
    out_shape=jax.ShapeDtypeStruct((_G, _D), jnp.float32),
    scratch_shapes=[
        pltpu.VMEM((_G, _D), jnp.float32),
        pltpu.VMEM((_G, 1), jnp.float32),
    ],
)


def kernel(x, edge_index, batch, emb,
           W0, b0, g0, be0, W1, b1, g1, be1, W2, b2, g2, be2):
    src2 = edge_index[0].reshape(_NC * _NS, _NCHUNK, _CH)
    dst2 = edge_index[1].reshape(_NC * _NS, _NCHUNK, _CH)
    zeros = jnp.zeros((_NP, _D), jnp.float32)

    deg2 = _deg(edge_index[1].reshape(_E, 1))
    degp = deg2.reshape(_NP, 1)[:_N]
    hh, dinv = _prep(x, degp, emb, W0)

    layers = ((b0, g0, be0, W1), (b1, g1, be1, W2), (b2, g2, be2, None))
    for (b, g, be, Wn) in layers:
        part = _sc_msg(hh, src2, dst2, zeros)
        a, st = _stats(part, hh, dinv, b.reshape(1, _D))
        if Wn is not None:
            hh = _next(a, st, g.reshape(1, _D), be.reshape(1, _D), Wn, dinv)
        else:
            out = _final(a, st, g.reshape(1, _D), be.reshape(1, _D),
                         batch.reshape(_N, 1))
    return out


# speedup vs baseline: 1.3427x; 1.3427x over previous
import jax
import jax.numpy as jnp
from jax import lax
from jax.experimental import pallas as pl
from jax.experimental.pallas import tpu as pltpu
from jax.experimental.pallas import tpu_sc as plsc

_N = 10000
_E = 320000
_D = 128
_G = 16
_EPS = 1e-5

_R = 1000
_NB = _N // _R

_NC = 2
_NS = 16
_CH = 125
_EPT = _E // (_NC * _NS)
_NCHUNK = _EPT // _CH
_NP = 10240
_RPT = _NP // _NS


def _msg_body(table, src2, dst2, zeros, out,
              idx_s, idx_d, rows, acc, sem):
    c = lax.axis_index("c")
    s = lax.axis_index("s")
    w = s * _NC + c
    pltpu.sync_copy(zeros.at[pl.ds(s * _RPT, _RPT)],
                    acc.at[pl.ds(s * _RPT, _RPT)])
    pltpu.sync_copy(src2.at[w], idx_s)
    pltpu.sync_copy(dst2.at[w], idx_d)
    plsc.subcore_barrier()

    def chunk(i, carry):
        pltpu.async_copy(table.at[idx_s.at[i]], rows, sem).wait()
        pltpu.sync_copy(rows, acc.at[idx_d.at[i]], add=True)
        return carry

    lax.fori_loop(0, _NCHUNK, chunk, 0)
    plsc.subcore_barrier()
    pltpu.sync_copy(acc.at[pl.ds(s * _RPT, _RPT)],
                    out.at[c, pl.ds(s * _RPT, _RPT)])


_sc_msg = pl.kernel(
    _msg_body,
    out_type=jax.ShapeDtypeStruct((_NC, _NP, _D), jnp.float32),
    mesh=plsc.VectorSubcoreMesh(core_axis_name="c", subcore_axis_name="s",
                                num_cores=_NC, num_subcores=_NS),
    scratch_types=[
        pltpu.VMEM((_NCHUNK, _CH), jnp.int32),
        pltpu.VMEM((_NCHUNK, _CH), jnp.int32),
        pltpu.VMEM((_CH, _D), jnp.float32),
        pltpu.VMEM_SHARED((_NP, _D), jnp.float32),
        pltpu.SemaphoreType.DMA,
    ],
)


_DW = 16


def _deg_body(dst2, zeros16, ones16, out, idx_d, rows, acc):
    c = lax.axis_index("c")
    s = lax.axis_index("s")
    w = s * _NC + c
    pltpu.sync_copy(zeros16.at[pl.ds(s * _RPT, _RPT)],
                    acc.at[pl.ds(s * _RPT, _RPT)])
    pltpu.sync_copy(dst2.at[w], idx_d)
    pltpu.sync_copy(ones16, rows)
    plsc.subcore_barrier()

    def chunk(i, carry):
        pltpu.sync_copy(rows, acc.at[idx_d.at[i]], add=True)
        return carry

    lax.fori_loop(0, _NCHUNK, chunk, 0)
    plsc.subcore_barrier()
    pltpu.sync_copy(acc.at[pl.ds(s * _RPT, _RPT)],
                    out.at[c, pl.ds(s * _RPT, _RPT)])


_sc_deg = pl.kernel(
    _deg_body,
    out_type=jax.ShapeDtypeStruct((_NC, _NP, _DW), jnp.float32),
    mesh=plsc.VectorSubcoreMesh(core_axis_name="c", subcore_axis_name="s",
                                num_cores=_NC, num_subcores=_NS),
    scratch_types=[
        pltpu.VMEM((_NCHUNK, _CH), jnp.int32),
        pltpu.VMEM((_CH, _DW), jnp.float32),
        pltpu.VMEM_SHARED((_NP, _DW), jnp.float32),
    ],
)


def _prep_body(x_ref, degp_ref, emb_ref, w0_ref, hh_ref, dinv_ref):
    deg = degp_ref[0, :, 0:1] + degp_ref[1, :, 0:1] + 1.0
    dinv = lax.rsqrt(deg)
    oh = (x_ref[...] == lax.broadcasted_iota(jnp.int32, (_R, _D), 1))
    ew = jnp.dot(emb_ref[...], w0_ref[...], preferred_element_type=jnp.float32,
                 precision=lax.Precision.HIGHEST)
    hh_ref[...] = jnp.dot(oh.astype(jnp.float32), ew,
                          preferred_element_type=jnp.float32,
                 precision=lax.Precision.HIGHEST) * dinv
    dinv_ref[...] = dinv


_prep = pl.pallas_call(
    _prep_body,
    grid=(_NB,),
    in_specs=[
        pl.BlockSpec((_R, 1), lambda i: (i, 0)),
        pl.BlockSpec((_NC, _R, _DW), lambda i: (0, i, 0)),
        pl.BlockSpec((_D, _D), lambda i: (0, 0)),
        pl.BlockSpec((_D, _D), lambda i: (0, 0)),
    ],
    out_specs=[
        pl.BlockSpec((_R, _D), lambda i: (i, 0)),
        pl.BlockSpec((_R, 1), lambda i: (i, 0)),
    ],
    out_shape=[
        jax.ShapeDtypeStruct((_N, _D), jnp.float32),
        jax.ShapeDtypeStruct((_N, 1), jnp.float32),
    ],
)


def _stats_body(part_ref, hh_ref, dinv_ref, b_ref, a_ref, st_ref):
    i = pl.program_id(0)
    act = (part_ref[0] + part_ref[1] + hh_ref[...]) * dinv_ref[...] + b_ref[...]
    a = jnp.maximum(act, 0.0)
    a_ref[...] = a

    @pl.when(i == 0)
    def _():
        st_ref[...] = jnp.zeros_like(st_ref)

    st_ref[0:1, :] += jnp.sum(a, axis=0, keepdims=True)
    st_ref[1:2, :] += jnp.sum(a * a, axis=0, keepdims=True)


_stats = pl.pallas_call(
    _stats_body,
    grid=(_NB,),
    in_specs=[
        pl.BlockSpec((_NC, _R, _D), lambda i: (0, i, 0)),
        pl.BlockSpec((_R, _D), lambda i: (i, 0)),
        pl.BlockSpec((_R, 1), lambda i: (i, 0)),
        pl.BlockSpec((1, _D), lambda i: (0, 0)),
    ],
    out_specs=[
        pl.BlockSpec((_R, _D), lambda i: (i, 0)),
        pl.BlockSpec((2, _D), lambda i: (0, 0)),
    ],
    out_shape=[
        jax.ShapeDtypeStruct((_N, _D), jnp.float32),
        jax.ShapeDtypeStruct((2, _D), jnp.float32),
    ],
)


def _bn(a_ref, st_ref, g_ref, be_ref):
    mu = st_ref[0:1, :] * (1.0 / _N)
    var = st_ref[1:2, :] * (1.0 / _N) - mu * mu
    return (a_ref[...] - mu) * lax.rsqrt(var + _EPS) * g_ref[...] + be_ref[...]


def _next_body(a_ref, st_ref, g_ref, be_ref, w_ref, dinv_ref, o_ref):
    hn = _bn(a_ref, st_ref, g_ref, be_ref)
    o_ref[...] = jnp.dot(hn, w_ref[...],
                         preferred_element_type=jnp.float32,
                 precision=lax.Precision.HIGHEST) * dinv_ref[...]


_next = pl.pallas_call(
    _next_body,
    grid=(_NB,),
    in_specs=[
        pl.BlockSpec((_R, _D), lambda i: (i, 0)),
        pl.BlockSpec((2, _D), lambda i: (0, 0)),
        pl.BlockSpec((1, _D), lambda i: (0, 0)),
        pl.BlockSpec((1, _D), lambda i: (0, 0)),
        pl.BlockSpec((_D, _D), lambda i: (0, 0)),
        pl.BlockSpec((_R, 1), lambda i: (i, 0)),
    ],
    out_specs=pl.BlockSpec((_R, _D), lambda i: (i, 0)),
    out_shape=jax.ShapeDtypeStruct((_N, _D), jnp.float32),
)


def _final_body(a_ref, st_ref, g_ref, be_ref, batch_ref, o_ref, sums, cnt):
    i = pl.program_id(0)
    hn = _bn(a_ref, st_ref, g_ref, be_ref)
    oh = (batch_ref[...] == lax.broadcasted_iota(jnp.int32, (_R, _G), 1)
          ).astype(jnp.float32)

    @pl.when(i == 0)
    def _():
        sums[...] = jnp.zeros_like(sums)
        cnt[...] = jnp.zeros_like(cnt)

    sums[...] += lax.dot_general(oh, hn, (((0,), (0,)), ((), ())),
                                 preferred_element_type=jnp.float32,
                                 precision=lax.Precision.HIGHEST)
    cnt[...] += jnp.sum(oh, axis=0)[:, None]

    @pl.when(i == _NB - 1)
    def _():
        o_ref[...] = sums[...] / jnp.maximum(cnt[...], 1.0)


_final = pl.pallas_call(
    _final_body,
    grid=(_NB,),
    in_specs=[
        pl.BlockSpec((_R, _D), lambda i: (i, 0)),
        pl.BlockSpec((2, _D), lambda i: (0, 0)),
        pl.BlockSpec((1, _D), lambda i: (0, 0)),
        pl.BlockSpec((1, _D), lambda i: (0, 0)),
        pl.BlockSpec((_R, 1), lambda i: (i, 0)),
    ],
    out_specs=pl.BlockSpec((_G, _D), lambda i: (0, 0)),
    out_shape=jax.ShapeDtypeStruct((_G, _D), jnp.float32),
    scratch_shapes=[
        pltpu.VMEM((_G, _D), jnp.float32),
        pltpu.VMEM((_G, 1), jnp.float32),
    ],
)


def kernel(x, edge_index, batch, emb,
           W0, b0, g0, be0, W1, b1, g1, be1, W2, b2, g2, be2):
    src2 = edge_index[0].reshape(_NC * _NS, _NCHUNK, _CH)
    dst2 = edge_index[1].reshape(_NC * _NS, _NCHUNK, _CH)
    zeros = jnp.zeros((_NP, _D), jnp.float32)

    degp = _sc_deg(dst2, jnp.zeros((_NP, _DW), jnp.float32),
                   jnp.ones((_CH, _DW), jnp.float32))
    hh, dinv = _prep(x, degp, emb, W0)

    layers = ((b0, g0, be0, W1), (b1, g1, be1, W2), (b2, g2, be2, None))
    for (b, g, be, Wn) in layers:
        part = _sc_msg(hh, src2, dst2, zeros)
        a, st = _stats(part, hh, dinv, b.reshape(1, _D))
        if Wn is not None:
            hh = _next(a, st, g.reshape(1, _D), be.reshape(1, _D), Wn, dinv)
        else:
            out = _final(a, st, g.reshape(1, _D), be.reshape(1, _D),
                         batch.reshape(_N, 1))
    return out
